# Initial kernel scaffold; baseline (speedup 1.0000x reference)
import jax
import jax.numpy as jnp
from jax.experimental import pallas as pl


def kernel(x, Wr, br, W1, b1, W2, b2):
    raise NotImplementedError("write your pallas kernel here")


# speedup vs baseline: 1.4147x; 1.4147x over previous
import functools

import jax
import jax.numpy as jnp
from jax.experimental import pallas as pl
from jax.experimental.pallas import tpu as pltpu

NUM_EXPERTS = 8
TOP_K = 2
D_MODEL = 1024
HIDDEN = 2048
TOKENS = 8192

BLK = 256
FLAT = TOKENS * TOP_K
NT = FLAT // BLK + NUM_EXPERTS
PTOT = NT * BLK

_INTERPRET = False


def _mlp_body(e_ref, xg_ref, w1_ref, b1_ref, w2_ref, b2_ref, sw_ref, o_ref):
    xb = xg_ref[...]
    w1 = w1_ref[0]
    h = jax.lax.dot_general(xb, w1, (((1,), (1,)), ((), ())),
                            preferred_element_type=jnp.float32)
    h = jax.nn.relu(h + b1_ref[0])
    w2 = w2_ref[0]
    o = jax.lax.dot_general(h, w2, (((1,), (1,)), ((), ())),
                            preferred_element_type=jnp.float32)
    o = jax.nn.sigmoid(o + b2_ref[0])
    o_ref[...] = o * sw_ref[0, 0][:, None]


@functools.partial(jax.jit, static_argnames=())
def _grouped_mlp(tile_expert, xg, W1, b1, W2, b2, slot_w):
    grid_spec = pltpu.PrefetchScalarGridSpec(
        num_scalar_prefetch=1,
        grid=(NT,),
        in_specs=[
            pl.BlockSpec((BLK, D_MODEL), lambda i, e: (i, 0)),
            pl.BlockSpec((1, HIDDEN, D_MODEL), lambda i, e: (e[i], 0, 0)),
            pl.BlockSpec((1, 1, HIDDEN), lambda i, e: (e[i], 0, 0)),
            pl.BlockSpec((1, D_MODEL, HIDDEN), lambda i, e: (e[i], 0, 0)),
            pl.BlockSpec((1, 1, D_MODEL), lambda i, e: (e[i], 0, 0)),
            pl.BlockSpec((1, 1, BLK), lambda i, e: (i, 0, 0)),
        ],
        out_specs=pl.BlockSpec((BLK, D_MODEL), lambda i, e: (i, 0)),
    )
    return pl.pallas_call(
        _mlp_body,
        grid_spec=grid_spec,
        out_shape=jax.ShapeDtypeStruct((PTOT, D_MODEL), jnp.float32),
        interpret=_INTERPRET,
    )(tile_expert, xg, W1, b1.reshape(NUM_EXPERTS, 1, HIDDEN), W2,
      b2.reshape(NUM_EXPERTS, 1, D_MODEL), slot_w.reshape(NT, 1, BLK))


def kernel(x, Wr, br, W1, b1, W2, b2):
    T = TOKENS
    logits = x @ Wr.T + br
    top_v, top_i = jax.lax.top_k(logits, TOP_K)
    top_w = jax.nn.softmax(top_v, axis=-1)

    e_flat = top_i.reshape(-1).astype(jnp.int32)
    onehot = (e_flat[:, None] == jnp.arange(NUM_EXPERTS, dtype=jnp.int32)[None, :]
              ).astype(jnp.int32)
    incl = jnp.cumsum(onehot, axis=0)
    counts = incl[-1]
    rank = jnp.take_along_axis(incl - onehot, e_flat[:, None], axis=1)[:, 0]
    padded = ((counts + BLK - 1) // BLK) * BLK
    offs = jnp.concatenate([jnp.zeros((1,), jnp.int32),
                            jnp.cumsum(padded)[:-1].astype(jnp.int32)])
    slot = offs[e_flat] + rank

    t_flat = (jnp.arange(FLAT, dtype=jnp.int32) // TOP_K)
    slot_token = jnp.zeros((PTOT,), jnp.int32).at[slot].set(t_flat)
    slot_w = jnp.zeros((PTOT,), jnp.float32).at[slot].set(top_w.reshape(-1))

    cum_end = jnp.cumsum(padded)
    tile_start = jnp.arange(NT, dtype=jnp.int32) * BLK
    tile_expert = jnp.sum(
        (tile_start[:, None] >= cum_end[None, :]).astype(jnp.int32), axis=1)
    tile_expert = jnp.minimum(tile_expert, NUM_EXPERTS - 1).astype(jnp.int32)

    xg = jnp.take(x, slot_token, axis=0)
    o_buf = _grouped_mlp(tile_expert, xg, W1, b1, W2, b2, slot_w)
    inv = slot.reshape(T, TOP_K)
    out = jnp.take(o_buf, inv[:, 0], axis=0) + jnp.take(o_buf, inv[:, 1], axis=0)
    return out
